# Initial kernel scaffold; baseline (speedup 1.0000x reference)
import jax
import jax.numpy as jnp
from jax.experimental import pallas as pl


def kernel(tokens, table):
    raise NotImplementedError("write your pallas kernel here")


# speedup vs baseline: 6.9936x; 6.9936x over previous
import functools

import jax
import jax.numpy as jnp
from jax import lax
from jax.experimental import pallas as pl
from jax.experimental.pallas import tpu as pltpu
from jax.experimental.pallas import tpu_sc as plsc

NC = 2
NS = 16
NW = NC * NS

GW = 128
K = 16


@functools.partial(jax.jit, static_argnums=(2, 3))
def _embedding_gather(tokens2d, table, rows_per_w, n_chunks):
    R = tokens2d.shape[0]
    D = table.shape[1]

    mesh = plsc.VectorSubcoreMesh(core_axis_name="c", subcore_axis_name="s")

    @functools.partial(
        pl.kernel,
        out_type=jax.ShapeDtypeStruct((R, GW, D), jnp.float32),
        mesh=mesh,
        scratch_types=[
            pltpu.VMEM((K, GW), jnp.int32),
            pltpu.VMEM((K, GW, D), jnp.float32),
            pltpu.SemaphoreType.DMA,
        ],
        compiler_params=pltpu.CompilerParams(use_tc_tiling_on_sc=False),
    )
    def k(tok_hbm, table_hbm, out_hbm, idx_v, rows_v, sem):
        wid = lax.axis_index("s") * NC + lax.axis_index("c")
        base = wid * rows_per_w

        def body(i, carry):
            start = base + i * K
            pltpu.sync_copy(tok_hbm.at[pl.ds(start, K)], idx_v)
            copies = []
            for j in range(K):
                copies.append(
                    pltpu.async_copy(
                        table_hbm.at[idx_v.at[j]], rows_v.at[j], sem
                    )
                )
            for c in copies:
                c.wait()
            pltpu.sync_copy(rows_v, out_hbm.at[pl.ds(start, K)])
            return carry

        lax.fori_loop(0, n_chunks, body, 0)

    return k(tokens2d, table)


def kernel(tokens, table):
    B0, S = tokens.shape
    V, D = table.shape
    B = B0 * S
    R = B // GW
    rows_per_w = R // NW
    n_chunks = rows_per_w // K
    assert R % NW == 0 and rows_per_w % K == 0

    tokens2d = tokens.reshape(R, GW)
    out = _embedding_gather(tokens2d, table, rows_per_w, n_chunks)
    return out.reshape(B0, S, D)


# speedup vs baseline: 7.0763x; 1.0118x over previous
import functools

import jax
import jax.numpy as jnp
from jax import lax
from jax.experimental import pallas as pl
from jax.experimental.pallas import tpu as pltpu
from jax.experimental.pallas import tpu_sc as plsc

NC = 2
NS = 16
NW = NC * NS

GW = 128
K = 10
NBUF = 2


@functools.partial(jax.jit, static_argnums=(2, 3))
def _embedding_gather(tokens2d, table, rows_per_w, n_chunks):
    R = tokens2d.shape[0]
    D = table.shape[1]

    mesh = plsc.VectorSubcoreMesh(core_axis_name="c", subcore_axis_name="s")

    @functools.partial(
        pl.kernel,
        out_type=jax.ShapeDtypeStruct((R, GW, D), jnp.float32),
        mesh=mesh,
        scratch_types=[
            pltpu.VMEM((NBUF, K, GW), jnp.int32),
            pltpu.VMEM((NBUF, K, GW, D), jnp.float32),
            pltpu.SemaphoreType.DMA((NBUF,)),
            pltpu.SemaphoreType.DMA((NBUF,)),
            pltpu.SemaphoreType.DMA((NBUF,)),
        ],
        compiler_params=pltpu.CompilerParams(use_tc_tiling_on_sc=False),
    )
    def k(tok_hbm, table_hbm, out_hbm, idx_v, rows_v, sem_i, sem_g, sem_o):
        wid = lax.axis_index("s") * NC + lax.axis_index("c")
        base = wid * rows_per_w

        def idx_copy(c, b):
            return pltpu.make_async_copy(
                tok_hbm.at[pl.ds(base + c * K, K)], idx_v.at[b], sem_i.at[b]
            )

        def out_copy(c, b):
            return pltpu.make_async_copy(
                rows_v.at[b], out_hbm.at[pl.ds(base + c * K, K)], sem_o.at[b]
            )

        for b in range(NBUF):
            idx_copy(b, b).start()

        def body(it, carry):
            for b in range(NBUF):
                c = it * NBUF + b
                idx_copy(c, b).wait()

                @pl.when(it > 0)
                def _():
                    out_copy(c - NBUF, b).wait()

                gathers = [
                    pltpu.async_copy(
                        table_hbm.at[idx_v.at[b].at[j]],
                        rows_v.at[b].at[j],
                        sem_g.at[b],
                    )
                    for j in range(K)
                ]
                for g in gathers:
                    g.wait()

                out_copy(c, b).start()

                @pl.when(c + NBUF < n_chunks)
                def _():
                    idx_copy(c + NBUF, b).start()

            return carry

        lax.fori_loop(0, n_chunks // NBUF, body, 0)

        for b in range(NBUF):
            out_copy(n_chunks - NBUF + b, b).wait()

    return k(tokens2d, table)


def kernel(tokens, table):
    B0, S = tokens.shape
    V, D = table.shape
    B = B0 * S
    R = B // GW
    rows_per_w = R // NW
    n_chunks = rows_per_w // K
    assert R % NW == 0 and rows_per_w % (K * NBUF) == 0

    tokens2d = tokens.reshape(R, GW)
    out = _embedding_gather(tokens2d, table, rows_per_w, n_chunks)
    return out.reshape(B0, S, D)


# speedup vs baseline: 7.3365x; 1.0368x over previous
import functools

import jax
import jax.numpy as jnp
from jax import lax
from jax.experimental import pallas as pl
from jax.experimental.pallas import tpu as pltpu
from jax.experimental.pallas import tpu_sc as plsc

NC = 2
NS = 16
NW = NC * NS

GW = 128
K = 8
NBUF = 2


@functools.partial(jax.jit, static_argnums=(2, 3))
def _embedding_gather(tokens2d, table, rows_per_w, n_chunks):
    R = tokens2d.shape[0]
    Vp, D = table.shape
    v_per_s = Vp // NS

    mesh = plsc.VectorSubcoreMesh(core_axis_name="c", subcore_axis_name="s")

    @functools.partial(
        pl.kernel,
        out_type=jax.ShapeDtypeStruct((R, GW, D), jnp.float32),
        mesh=mesh,
        scratch_types=[
            pltpu.VMEM((NBUF, K, GW), jnp.int32),
            pltpu.VMEM((NBUF, K, GW, D), jnp.float32),
            pltpu.VMEM_SHARED((Vp, D), jnp.float32),
            pltpu.SemaphoreType.DMA((NBUF,)),
            pltpu.SemaphoreType.DMA((NBUF,)),
            pltpu.SemaphoreType.DMA((NBUF,)),
        ],
        compiler_params=pltpu.CompilerParams(use_tc_tiling_on_sc=False),
    )
    def k(tok_hbm, table_hbm, out_hbm, idx_v, rows_v, table_sh,
          sem_i, sem_g, sem_o):
        sid = lax.axis_index("s")
        wid = sid * NC + lax.axis_index("c")
        base = wid * rows_per_w

        pltpu.sync_copy(
            table_hbm.at[pl.ds(sid * v_per_s, v_per_s)],
            table_sh.at[pl.ds(sid * v_per_s, v_per_s)],
        )
        plsc.subcore_barrier()

        def idx_copy(c, b):
            return pltpu.make_async_copy(
                tok_hbm.at[pl.ds(base + c * K, K)], idx_v.at[b], sem_i.at[b]
            )

        def out_copy(c, b):
            return pltpu.make_async_copy(
                rows_v.at[b], out_hbm.at[pl.ds(base + c * K, K)], sem_o.at[b]
            )

        for b in range(NBUF):
            idx_copy(b, b).start()

        def body(it, carry):
            for b in range(NBUF):
                c = it * NBUF + b
                idx_copy(c, b).wait()

                @pl.when(it > 0)
                def _():
                    out_copy(c - NBUF, b).wait()

                gathers = [
                    pltpu.async_copy(
                        table_sh.at[idx_v.at[b].at[j]],
                        rows_v.at[b].at[j],
                        sem_g.at[b],
                    )
                    for j in range(K)
                ]
                for g in gathers:
                    g.wait()

                out_copy(c, b).start()

                @pl.when(c + NBUF < n_chunks)
                def _():
                    idx_copy(c + NBUF, b).start()

            return carry

        lax.fori_loop(0, n_chunks // NBUF, body, 0)

        for b in range(NBUF):
            out_copy(n_chunks - NBUF + b, b).wait()

    return k(tokens2d, table)


def kernel(tokens, table):
    B0, S = tokens.shape
    V, D = table.shape
    B = B0 * S
    R = B // GW
    rows_per_w = R // NW
    n_chunks = rows_per_w // K
    assert R % NW == 0 and rows_per_w % (K * NBUF) == 0

    Vp = ((V + 8 * NS - 1) // (8 * NS)) * (8 * NS)
    table_p = jnp.pad(table, ((0, Vp - V), (0, 0)))

    tokens2d = tokens.reshape(R, GW)
    out = _embedding_gather(tokens2d, table_p, rows_per_w, n_chunks)
    return out.reshape(B0, S, D)
